# baseline (speedup 1.0000x reference)
import functools

import jax
import jax.numpy as jnp
from jax.experimental import pallas as pl

TOPK = 32


def _fused_kernel(x_ref, c_ref, sims_ref, idx_ref, am_ref, *, inv_sqrt_d):
    xb = x_ref[0]
    c = c_ref[...]
    logits_t = jax.lax.dot_general(
        c, xb, (((1,), (1,)), ((), ())),
        preferred_element_type=jnp.float32,
    ) * inv_sqrt_d
    m = jnp.max(logits_t, axis=0, keepdims=True)
    e = jnp.exp(logits_t - m)
    probs_t = e / jnp.sum(e, axis=0, keepdims=True)
    am_ref[...] = probs_t.T

    iota = jax.lax.broadcasted_iota(jnp.int32, probs_t.shape, 0)
    vals = probs_t
    sims_rows = []
    idx_rows = []
    for _ in range(TOPK):
        v, ix = vals, iota
        while v.shape[0] > 1:
            h = v.shape[0] // 2
            a_v, b_v = v[:h], v[h:]
            take = a_v >= b_v
            v = jnp.maximum(a_v, b_v)
            ix = jnp.where(take, ix[:h], ix[h:])
        sims_rows.append(v)
        idx_rows.append(ix)
        vals = jnp.where(iota == ix, -1.0, vals)

    for start in (0, 1, 0):
        for j in range(start, TOPK - 1, 2):
            tie = sims_rows[j] == sims_rows[j + 1]
            lo = jnp.minimum(idx_rows[j], idx_rows[j + 1])
            hi = jnp.maximum(idx_rows[j], idx_rows[j + 1])
            idx_rows[j] = jnp.where(tie, lo, idx_rows[j])
            idx_rows[j + 1] = jnp.where(tie, hi, idx_rows[j + 1])

    sims_ref[0] = jnp.concatenate(sims_rows, axis=0).T
    idx_ref[0] = jnp.concatenate(idx_rows, axis=0).T


@jax.jit
def kernel(x, centroids):
    B, S, D = x.shape
    K = centroids.shape[0]
    S_blk = 512
    grid = (B, S // S_blk)
    x = x.astype(jnp.bfloat16)
    centroids = centroids.astype(jnp.bfloat16)
    body = functools.partial(_fused_kernel, inv_sqrt_d=1.0 / (D ** 0.5))
    sims, indices, amatrix_r = pl.pallas_call(
        body,
        grid=grid,
        in_specs=[
            pl.BlockSpec((1, S_blk, D), lambda b, s: (b, s, 0)),
            pl.BlockSpec((K, D), lambda b, s: (0, 0)),
        ],
        out_specs=[
            pl.BlockSpec((1, S_blk, TOPK), lambda b, s: (b, s, 0)),
            pl.BlockSpec((1, S_blk, TOPK), lambda b, s: (b, s, 0)),
            pl.BlockSpec((S_blk, K), lambda b, s: (s, b)),
        ],
        out_shape=[
            jax.ShapeDtypeStruct((B, S, TOPK), jnp.float32),
            jax.ShapeDtypeStruct((B, S, TOPK), jnp.int32),
            jax.ShapeDtypeStruct((S, B * K), jnp.float32),
        ],
    )(x, centroids)
    return sims, indices, amatrix_r


# speedup vs baseline: 1.2052x; 1.2052x over previous
import functools

import jax
import jax.numpy as jnp
from jax.experimental import pallas as pl
from jax.experimental.pallas import tpu as pltpu

TOPK = 32


def _fused_kernel(x_ref, c_ref, sims_ref, idx_ref, am_ref, *, inv_sqrt_d):
    xb = x_ref[0]
    c = c_ref[...]
    logits_t = jax.lax.dot_general(
        c.astype(jnp.bfloat16), xb.astype(jnp.bfloat16),
        (((1,), (1,)), ((), ())),
        preferred_element_type=jnp.float32,
    ) * inv_sqrt_d
    m = jnp.max(logits_t, axis=0, keepdims=True)
    e = jnp.exp(logits_t - m)
    probs_t = e / jnp.sum(e, axis=0, keepdims=True)
    am_ref[...] = probs_t.T

    iota = jax.lax.broadcasted_iota(jnp.int32, probs_t.shape, 0)
    vals = probs_t
    sims_rows = []
    idx_rows = []
    for _ in range(TOPK):
        v, ix = vals, iota
        while v.shape[0] > 1:
            h = v.shape[0] // 2
            a_v, b_v = v[:h], v[h:]
            take = a_v >= b_v
            v = jnp.maximum(a_v, b_v)
            ix = jnp.where(take, ix[:h], ix[h:])
        sims_rows.append(v)
        idx_rows.append(ix)
        vals = jnp.where(iota == ix, -1.0, vals)

    for start in (0, 1, 0):
        for j in range(start, TOPK - 1, 2):
            tie = sims_rows[j] == sims_rows[j + 1]
            lo = jnp.minimum(idx_rows[j], idx_rows[j + 1])
            hi = jnp.maximum(idx_rows[j], idx_rows[j + 1])
            idx_rows[j] = jnp.where(tie, lo, idx_rows[j])
            idx_rows[j + 1] = jnp.where(tie, hi, idx_rows[j + 1])

    sims_ref[0] = jnp.concatenate(sims_rows, axis=0).T
    idx_ref[0] = jnp.concatenate(idx_rows, axis=0).T


@jax.jit
def kernel(x, centroids):
    B, S, D = x.shape
    K = centroids.shape[0]
    S_blk = 512
    grid = (B, S // S_blk)
    body = functools.partial(_fused_kernel, inv_sqrt_d=1.0 / (D ** 0.5))
    sims, indices, amatrix_r = pl.pallas_call(
        body,
        grid=grid,
        in_specs=[
            pl.BlockSpec((1, S_blk, D), lambda b, s: (b, s, 0)),
            pl.BlockSpec((K, D), lambda b, s: (0, 0)),
        ],
        out_specs=[
            pl.BlockSpec((1, S_blk, TOPK), lambda b, s: (b, s, 0)),
            pl.BlockSpec((1, S_blk, TOPK), lambda b, s: (b, s, 0)),
            pl.BlockSpec((S_blk, K), lambda b, s: (s, b)),
        ],
        out_shape=[
            jax.ShapeDtypeStruct((B, S, TOPK), jnp.float32),
            jax.ShapeDtypeStruct((B, S, TOPK), jnp.int32),
            jax.ShapeDtypeStruct((S, B * K), jnp.float32),
        ],
        compiler_params=pltpu.CompilerParams(
            dimension_semantics=("parallel", "parallel")),
    )(x, centroids)
    return sims, indices, amatrix_r


# speedup vs baseline: 1.2286x; 1.0194x over previous
import functools

import jax
import jax.numpy as jnp
from jax.experimental import pallas as pl
from jax.experimental.pallas import tpu as pltpu

TOPK = 32


def _fused_kernel(x_ref, c_ref, sims_ref, idx_ref, am_ref, *, inv_sqrt_d):
    xb = x_ref[0]
    c = c_ref[...]
    logits_t = jax.lax.dot_general(
        c.astype(jnp.bfloat16), xb.astype(jnp.bfloat16),
        (((1,), (1,)), ((), ())),
        preferred_element_type=jnp.float32,
    ) * inv_sqrt_d
    m = jnp.max(logits_t, axis=0, keepdims=True)
    e = jnp.exp(logits_t - m)
    probs_t = e / jnp.sum(e, axis=0, keepdims=True)
    am_ref[...] = probs_t.T

    iota = jax.lax.broadcasted_iota(jnp.int32, probs_t.shape, 0)
    vals = probs_t
    sims_rows = []
    idx_rows = []
    for _ in range(TOPK):
        v, ix = vals, iota
        while v.shape[0] > 1:
            h = v.shape[0] // 2
            a_v, b_v = v[:h], v[h:]
            take = a_v >= b_v
            v = jnp.maximum(a_v, b_v)
            ix = jnp.where(take, ix[:h], ix[h:])
        sims_rows.append(v)
        idx_rows.append(ix)
        vals = jnp.where(iota == ix, -1.0, vals)

    for start in (0, 1, 0):
        for j in range(start, TOPK - 1, 2):
            tie = sims_rows[j] == sims_rows[j + 1]
            lo = jnp.minimum(idx_rows[j], idx_rows[j + 1])
            hi = jnp.maximum(idx_rows[j], idx_rows[j + 1])
            idx_rows[j] = jnp.where(tie, lo, idx_rows[j])
            idx_rows[j + 1] = jnp.where(tie, hi, idx_rows[j + 1])

    sims_ref[0] = jnp.concatenate(sims_rows, axis=0).T
    idx_ref[0] = jnp.concatenate(idx_rows, axis=0).T


@jax.jit
def kernel(x, centroids):
    B, S, D = x.shape
    K = centroids.shape[0]
    S_blk = 1024
    grid = (B, S // S_blk)
    body = functools.partial(_fused_kernel, inv_sqrt_d=1.0 / (D ** 0.5))
    sims, indices, amatrix_r = pl.pallas_call(
        body,
        grid=grid,
        in_specs=[
            pl.BlockSpec((1, S_blk, D), lambda b, s: (b, s, 0)),
            pl.BlockSpec((K, D), lambda b, s: (0, 0)),
        ],
        out_specs=[
            pl.BlockSpec((1, S_blk, TOPK), lambda b, s: (b, s, 0)),
            pl.BlockSpec((1, S_blk, TOPK), lambda b, s: (b, s, 0)),
            pl.BlockSpec((S_blk, K), lambda b, s: (s, b)),
        ],
        out_shape=[
            jax.ShapeDtypeStruct((B, S, TOPK), jnp.float32),
            jax.ShapeDtypeStruct((B, S, TOPK), jnp.int32),
            jax.ShapeDtypeStruct((S, B * K), jnp.float32),
        ],
        compiler_params=pltpu.CompilerParams(
            dimension_semantics=("parallel", "parallel")),
    )(x, centroids)
    return sims, indices, amatrix_r


# speedup vs baseline: 1.7850x; 1.4529x over previous
import functools

import jax
import jax.numpy as jnp
from jax.experimental import pallas as pl
from jax.experimental.pallas import tpu as pltpu

TOPK = 32


def _fused_kernel(x_ref, c_ref, sims_ref, idx_ref, am_ref, *, inv_sqrt_d):
    xb = x_ref[0]
    c = c_ref[...]
    logits_t = jax.lax.dot_general(
        c.astype(jnp.bfloat16), xb.astype(jnp.bfloat16),
        (((1,), (1,)), ((), ())),
        preferred_element_type=jnp.float32,
    ) * inv_sqrt_d
    m = jnp.max(logits_t, axis=0, keepdims=True)
    e = jnp.exp(logits_t - m)
    probs_t = e / jnp.sum(e, axis=0, keepdims=True)
    am_ref[...] = probs_t.T

    kdim, sb = probs_t.shape
    ng = kdim // 8
    iota = jax.lax.broadcasted_iota(jnp.int32, probs_t.shape, 0)
    mem = [probs_t[ng * i:ng * (i + 1)] for i in range(8)]
    mid = [iota[ng * i:ng * (i + 1)] for i in range(8)]
    net = [(0, 1), (2, 3), (4, 5), (6, 7),
           (0, 2), (1, 3), (4, 6), (5, 7),
           (1, 2), (5, 6),
           (0, 4), (1, 5), (2, 6), (3, 7),
           (2, 4), (3, 5),
           (1, 2), (3, 4), (5, 6)]
    for a, b in net:
        ta, tb = mem[a], mem[b]
        ia_, ib_ = mid[a], mid[b]
        take = ta >= tb
        mem[a] = jnp.maximum(ta, tb)
        mem[b] = jnp.minimum(ta, tb)
        mid[a] = jnp.where(take, ia_, ib_)
        mid[b] = jnp.where(take, ib_, ia_)
    giota = jax.lax.broadcasted_iota(jnp.int32, mem[0].shape, 0)
    sims_rows = []
    idx_rows = []
    for _ in range(TOPK):
        v, ix, gx = mem[0], mid[0], giota
        while v.shape[0] > 1:
            h = v.shape[0] // 2
            take = v[:h] >= v[h:]
            ix = jnp.where(take, ix[:h], ix[h:])
            gx = jnp.where(take, gx[:h], gx[h:])
            v = jnp.maximum(v[:h], v[h:])
        sims_rows.append(v)
        idx_rows.append(ix)
        w = giota == gx
        for k in range(7):
            mem[k] = jnp.where(w, mem[k + 1], mem[k])
            mid[k] = jnp.where(w, mid[k + 1], mid[k])
        mem[7] = jnp.where(w, -1.0, mem[7])

    for start in (0, 1, 0):
        for j in range(start, TOPK - 1, 2):
            tie = sims_rows[j] == sims_rows[j + 1]
            lo = jnp.minimum(idx_rows[j], idx_rows[j + 1])
            hi = jnp.maximum(idx_rows[j], idx_rows[j + 1])
            idx_rows[j] = jnp.where(tie, lo, idx_rows[j])
            idx_rows[j + 1] = jnp.where(tie, hi, idx_rows[j + 1])

    sims_ref[0] = jnp.concatenate(sims_rows, axis=0).T
    idx_ref[0] = jnp.concatenate(idx_rows, axis=0).T


@jax.jit
def kernel(x, centroids):
    B, S, D = x.shape
    K = centroids.shape[0]
    S_blk = 1024
    grid = (B, S // S_blk)
    body = functools.partial(_fused_kernel, inv_sqrt_d=1.0 / (D ** 0.5))
    sims, indices, amatrix_r = pl.pallas_call(
        body,
        grid=grid,
        in_specs=[
            pl.BlockSpec((1, S_blk, D), lambda b, s: (b, s, 0)),
            pl.BlockSpec((K, D), lambda b, s: (0, 0)),
        ],
        out_specs=[
            pl.BlockSpec((1, S_blk, TOPK), lambda b, s: (b, s, 0)),
            pl.BlockSpec((1, S_blk, TOPK), lambda b, s: (b, s, 0)),
            pl.BlockSpec((S_blk, K), lambda b, s: (s, b)),
        ],
        out_shape=[
            jax.ShapeDtypeStruct((B, S, TOPK), jnp.float32),
            jax.ShapeDtypeStruct((B, S, TOPK), jnp.int32),
            jax.ShapeDtypeStruct((S, B * K), jnp.float32),
        ],
        compiler_params=pltpu.CompilerParams(
            dimension_semantics=("parallel", "parallel")),
    )(x, centroids)
    return sims, indices, amatrix_r


# speedup vs baseline: 1.8021x; 1.0095x over previous
import functools

import jax
import jax.numpy as jnp
from jax.experimental import pallas as pl
from jax.experimental.pallas import tpu as pltpu

TOPK = 32


def _fused_kernel(x_ref, c_ref, sims_ref, idx_ref, am_ref, *, inv_sqrt_d):
    xb = x_ref[0]
    c = c_ref[...]
    logits_t = jax.lax.dot_general(
        c.astype(jnp.bfloat16), xb.astype(jnp.bfloat16),
        (((1,), (1,)), ((), ())),
        preferred_element_type=jnp.float32,
    ) * inv_sqrt_d
    m = jnp.max(logits_t, axis=0, keepdims=True)
    e = jnp.exp(logits_t - m)
    probs_t = e / jnp.sum(e, axis=0, keepdims=True)
    am_ref[...] = probs_t.T

    kdim, sb = probs_t.shape
    ng = kdim // 8
    iota = jax.lax.broadcasted_iota(jnp.int32, probs_t.shape, 0)
    mem = [probs_t[ng * i:ng * (i + 1)] for i in range(8)]
    mid = [iota[ng * i:ng * (i + 1)] for i in range(8)]
    net = [(0, 1), (2, 3), (4, 5), (6, 7),
           (0, 2), (1, 3), (4, 6), (5, 7),
           (1, 2), (5, 6),
           (0, 4), (1, 5), (2, 6), (3, 7),
           (2, 4), (3, 5),
           (1, 2), (3, 4), (5, 6)]
    for a, b in net:
        ta, tb = mem[a], mem[b]
        ia_, ib_ = mid[a], mid[b]
        take = ta >= tb
        mem[a] = jnp.maximum(ta, tb)
        mem[b] = jnp.minimum(ta, tb)
        mid[a] = jnp.where(take, ia_, ib_)
        mid[b] = jnp.where(take, ib_, ia_)
    giota = jax.lax.broadcasted_iota(jnp.int32, mem[0].shape, 0)
    sims_rows = []
    idx_rows = []
    for _ in range(TOPK):
        v, ix = mem[0], mid[0]
        while v.shape[0] > 1:
            h = v.shape[0] // 2
            take = v[:h] >= v[h:]
            ix = jnp.where(take, ix[:h], ix[h:])
            v = jnp.maximum(v[:h], v[h:])
        sims_rows.append(v)
        idx_rows.append(ix)
        w = giota == jax.lax.bitwise_and(ix, ng - 1)
        for k in range(7):
            mem[k] = jnp.where(w, mem[k + 1], mem[k])
            mid[k] = jnp.where(w, mid[k + 1], mid[k])
        mem[7] = jnp.where(w, -1.0, mem[7])

    for start in (0, 1, 0):
        for j in range(start, TOPK - 1, 2):
            tie = sims_rows[j] == sims_rows[j + 1]
            lo = jnp.minimum(idx_rows[j], idx_rows[j + 1])
            hi = jnp.maximum(idx_rows[j], idx_rows[j + 1])
            idx_rows[j] = jnp.where(tie, lo, idx_rows[j])
            idx_rows[j + 1] = jnp.where(tie, hi, idx_rows[j + 1])

    sims_ref[0] = jnp.concatenate(sims_rows, axis=0).T
    idx_ref[0] = jnp.concatenate(idx_rows, axis=0).T


@jax.jit
def kernel(x, centroids):
    B, S, D = x.shape
    K = centroids.shape[0]
    S_blk = 1024
    grid = (B, S // S_blk)
    body = functools.partial(_fused_kernel, inv_sqrt_d=1.0 / (D ** 0.5))
    sims, indices, amatrix_r = pl.pallas_call(
        body,
        grid=grid,
        in_specs=[
            pl.BlockSpec((1, S_blk, D), lambda b, s: (b, s, 0)),
            pl.BlockSpec((K, D), lambda b, s: (0, 0)),
        ],
        out_specs=[
            pl.BlockSpec((1, S_blk, TOPK), lambda b, s: (b, s, 0)),
            pl.BlockSpec((1, S_blk, TOPK), lambda b, s: (b, s, 0)),
            pl.BlockSpec((S_blk, K), lambda b, s: (s, b)),
        ],
        out_shape=[
            jax.ShapeDtypeStruct((B, S, TOPK), jnp.float32),
            jax.ShapeDtypeStruct((B, S, TOPK), jnp.int32),
            jax.ShapeDtypeStruct((S, B * K), jnp.float32),
        ],
        compiler_params=pltpu.CompilerParams(
            dimension_semantics=("parallel", "parallel")),
    )(x, centroids)
    return sims, indices, amatrix_r
